# Initial kernel scaffold; baseline (speedup 1.0000x reference)
import jax
import jax.numpy as jnp
from jax.experimental import pallas as pl


def kernel(x, batch, W1, b1, W2, b2):
    raise NotImplementedError("write your pallas kernel here")


# speedup vs baseline: 4.4700x; 4.4700x over previous
import jax
import jax.numpy as jnp
from jax import lax
from jax.experimental import pallas as pl
from jax.experimental.pallas import tpu as pltpu

R = 128
C = 512
BIG = 1e30
IDX_BIG = 1e9

HIGH = lax.Precision.HIGHEST


def _dot(a, b, dims, precision=HIGH):
    return lax.dot_general(a, b, (dims, ((), ())),
                           precision=precision,
                           preferred_element_type=jnp.float32)


def _knn_proj_kernel(starts_ref, ncr_ref, xbf_ref, xf_ref, sqr_ref, sqc_ref,
                     rs_ref, re_ref, w1m_ref, w1b_ref, b1_ref,
                     topi_ref, a_ref, b_ref, dist_scr):
    blk = pl.program_id(0)
    start = starts_ref[blk]
    ncr = ncr_ref[blk]

    xr_b = xbf_ref[pl.ds(pl.multiple_of(blk * R, R), R), :]
    rs = rs_ref[:]
    re = re_ref[:]
    sqr = sqr_ref[:]

    xr = xf_ref[pl.ds(pl.multiple_of(blk * R, R), R), :]
    a_ref[:] = _dot(xr, w1m_ref[:], ((1,), (0,))) + b1_ref[:]
    b_ref[:] = _dot(xr, w1b_ref[:], ((1,), (0,)))

    lane = lax.broadcasted_iota(jnp.int32, (1, C), 1).astype(jnp.float32)

    def fill(c, _):
        off = start + c * C
        xc_c = xbf_ref[pl.ds(pl.multiple_of(off, C), C), :]
        d0 = _dot(xr_b, xc_c, ((1,), (1,)), precision=None)
        sqc = sqc_ref[start // C + c, 0:1, :]
        d = (sqr + sqc) - 2.0 * d0
        gi = off.astype(jnp.float32) + lane
        valid = (gi >= rs) & (gi < re)
        dist_scr[:, pl.ds(pl.multiple_of(c * C, C), C)] = jnp.where(valid, d, BIG)
        return 0

    lax.fori_loop(0, ncr, fill, 0, unroll=False)

    m_prev = jnp.full((R, 1), -jnp.inf, jnp.float32)
    i_prev = jnp.full((R, 1), -1.0, jnp.float32)
    cols = []
    for _ in range(20):
        def scan(c, carry):
            bv, bi = carry
            v = dist_scr[:, pl.ds(pl.multiple_of(c * C, C), C)]
            gi = (start + c * C).astype(jnp.float32) + lane
            ok = (v > m_prev) | ((v == m_prev) & (gi > i_prev))
            vv = jnp.where(ok, v, jnp.inf)
            cm = jnp.min(vv, axis=1, keepdims=True)
            ci = jnp.min(jnp.where(vv == cm, gi, IDX_BIG), axis=1,
                         keepdims=True)
            take = (cm < bv) | ((cm == bv) & (ci < bi))
            return jnp.where(take, cm, bv), jnp.where(take, ci, bi)

        m_prev, i_prev = lax.fori_loop(
            0, ncr, scan,
            (jnp.full((R, 1), jnp.inf, jnp.float32),
             jnp.full((R, 1), IDX_BIG, jnp.float32)),
            unroll=False)
        cols.append(i_prev)

    topi_ref[:] = jnp.concatenate(cols, axis=1)


def _edge_mlp_kernel(starts_ref, ncr_ref, topi_ref, a_ref, bhi_ref, blo_ref,
                     w2_ref, b2_ref, out_ref):
    blk = pl.program_id(0)
    start = starts_ref[blk]
    ncr = ncr_ref[blk]

    a = a_ref[:]
    lane = lax.broadcasted_iota(jnp.int32, (1, C), 1).astype(jnp.float32)

    out = jnp.full((R, 128), -jnp.inf, jnp.float32)
    for k in range(20):
        idxk = topi_ref[:, k:k + 1]

        def gath(c, g):
            off = start + c * C
            gi = off.astype(jnp.float32) + lane
            oh = (idxk == gi).astype(jnp.bfloat16)
            bh = bhi_ref[pl.ds(pl.multiple_of(off, C), C), :]
            bl = blo_ref[pl.ds(pl.multiple_of(off, C), C), :]
            g = g + _dot(oh, bh, ((1,), (0,)), precision=None)
            g = g + _dot(oh, bl, ((1,), (0,)), precision=None)
            return g

        gk = lax.fori_loop(0, ncr, gath, jnp.zeros((R, 64), jnp.float32),
                           unroll=False)
        h = jnp.maximum(a + gk, 0.0)
        out = jnp.maximum(out, _dot(h, w2_ref[:], ((1,), (0,))))

    out_ref[:] = out + b2_ref[:]


def kernel(x, batch, W1, b1, W2, b2, _debug_parts=False):
    n, d = x.shape
    n_pad = ((n + C - 1) // C) * C
    nb = n_pad // R

    pad_id = batch[-1] + 1
    x_pad = jnp.pad(x, ((0, n_pad - n), (0, 0)))
    batch_pad = jnp.concatenate(
        [batch, jnp.full((n_pad - n,), pad_id, batch.dtype)])

    x_bf = x_pad.astype(jnp.bfloat16)
    sq = jnp.sum(x_pad * x_pad, axis=1)
    sqr_f = sq[:, None]
    nch_total = n_pad // C
    sqc3 = jnp.broadcast_to(sq.reshape(nch_total, 1, C),
                            (nch_total, 8, C))

    rs_all = jnp.searchsorted(batch_pad, batch_pad, side='left')
    re_all = jnp.searchsorted(batch_pad, batch_pad, side='right')
    start_blk = rs_all.reshape(nb, R)[:, 0].astype(jnp.int32)
    end_blk = re_all.reshape(nb, R)[:, -1].astype(jnp.int32)
    start_al = (start_blk // C) * C
    ncr = (end_blk - start_al + C - 1) // C

    rs_f = rs_all.astype(jnp.float32)[:, None]
    re_f = re_all.astype(jnp.float32)[:, None]

    W1m = W1[:d] - W1[d:]
    W1b = W1[d:]
    b1r = b1[None, :]
    b2r = b2[None, :]

    smem = pl.BlockSpec(memory_space=pltpu.SMEM)
    full = pl.BlockSpec(memory_space=pltpu.VMEM)

    grid = (nb,)
    topi, A, B = pl.pallas_call(
        _knn_proj_kernel,
        grid=grid,
        in_specs=[
            smem, smem,
            full, full,
            pl.BlockSpec((R, 1), lambda b: (b, 0)),
            full,
            pl.BlockSpec((R, 1), lambda b: (b, 0)),
            pl.BlockSpec((R, 1), lambda b: (b, 0)),
            full, full, full,
        ],
        out_specs=[
            pl.BlockSpec((R, 20), lambda b: (b, 0)),
            pl.BlockSpec((R, 64), lambda b: (b, 0)),
            pl.BlockSpec((R, 64), lambda b: (b, 0)),
        ],
        out_shape=[
            jax.ShapeDtypeStruct((n_pad, 20), jnp.float32),
            jax.ShapeDtypeStruct((n_pad, 64), jnp.float32),
            jax.ShapeDtypeStruct((n_pad, 64), jnp.float32),
        ],
        scratch_shapes=[pltpu.VMEM((R, n_pad), jnp.float32)],
    )(start_al, ncr, x_bf, x_pad, sqr_f, sqc3, rs_f, re_f, W1m, W1b, b1r)

    Bhi = B.astype(jnp.bfloat16)
    Blo = (B - Bhi.astype(jnp.float32)).astype(jnp.bfloat16)

    out = pl.pallas_call(
        _edge_mlp_kernel,
        grid=grid,
        in_specs=[
            smem, smem,
            pl.BlockSpec((R, 20), lambda b: (b, 0)),
            pl.BlockSpec((R, 64), lambda b: (b, 0)),
            full, full, full, full,
        ],
        out_specs=pl.BlockSpec((R, 128), lambda b: (b, 0)),
        out_shape=jax.ShapeDtypeStruct((n_pad, 128), jnp.float32),
    )(start_al, ncr, topi, A, Bhi, Blo, W2, b2r)

    if _debug_parts:
        return out[:n], topi, A, B
    return out[:n]


# speedup vs baseline: 5.4764x; 1.2251x over previous
import jax
import jax.numpy as jnp
from jax import lax
from jax.experimental import pallas as pl
from jax.experimental.pallas import tpu as pltpu

R = 128
C = 512
BIG = 1e30
IDX_BIG = 1e9

HIGH = lax.Precision.HIGHEST


def _dot(a, b, dims, precision=HIGH):
    return lax.dot_general(a, b, (dims, ((), ())),
                           precision=precision,
                           preferred_element_type=jnp.float32)


def _knn_proj_kernel(starts_ref, ncr_ref, xbf_ref, xf_ref, sqr_ref, sqc_ref,
                     rs_ref, re_ref, w1m_ref, w1b_ref, b1_ref,
                     topi_ref, a_ref, b_ref, dist_scr):
    blk = pl.program_id(0)
    start = starts_ref[blk]
    ncr = ncr_ref[blk]

    xr_b = xbf_ref[pl.ds(pl.multiple_of(blk * R, R), R), :]
    rs = rs_ref[:]
    re = re_ref[:]
    sqr = sqr_ref[:]

    xr = xf_ref[pl.ds(pl.multiple_of(blk * R, R), R), :]
    a_ref[:] = _dot(xr, w1m_ref[:], ((1,), (0,))) + b1_ref[:]
    b_ref[:] = _dot(xr, w1b_ref[:], ((1,), (0,)))

    lane = lax.broadcasted_iota(jnp.int32, (1, C), 1).astype(jnp.float32)

    def fill(c, _):
        off = start + c * C
        xc_c = xbf_ref[pl.ds(pl.multiple_of(off, C), C), :]
        d0 = _dot(xr_b, xc_c, ((1,), (1,)), precision=None)
        sqc = sqc_ref[start // C + c, 0:1, :]
        d = (sqr + sqc) - 2.0 * d0
        gi = off.astype(jnp.float32) + lane
        valid = (gi >= rs) & (gi < re)
        dist_scr[:, pl.ds(pl.multiple_of(c * C, C), C)] = jnp.where(valid, d, BIG)
        return 0

    lax.fori_loop(0, ncr, fill, 0, unroll=False)

    m_prev = jnp.full((R, 1), -jnp.inf, jnp.float32)
    i_prev = jnp.full((R, 1), -1.0, jnp.float32)
    cols = []
    for _ in range(20):
        def scan(c, carry):
            bv, bi = carry
            v = dist_scr[:, pl.ds(pl.multiple_of(c * C, C), C)]
            gi = (start + c * C).astype(jnp.float32) + lane
            ok = (v > m_prev) | ((v == m_prev) & (gi > i_prev))
            vv = jnp.where(ok, v, jnp.inf)
            cm = jnp.min(vv, axis=1, keepdims=True)
            ci = jnp.min(jnp.where(vv == cm, gi, IDX_BIG), axis=1,
                         keepdims=True)
            take = (cm < bv) | ((cm == bv) & (ci < bi))
            return jnp.where(take, cm, bv), jnp.where(take, ci, bi)

        m_prev, i_prev = lax.fori_loop(
            0, ncr, scan,
            (jnp.full((R, 1), jnp.inf, jnp.float32),
             jnp.full((R, 1), IDX_BIG, jnp.float32)),
            unroll=False)
        cols.append(i_prev)

    topi_ref[:] = jnp.concatenate(cols, axis=1)


def _edge_mlp_kernel(starts_ref, ncr_ref, topi_ref, a_ref, bhi_ref, blo_ref,
                     w2_ref, b2_ref, out_ref):
    blk = pl.program_id(0)
    start = starts_ref[blk]
    ncr = ncr_ref[blk]

    a = a_ref[:]
    lane = lax.broadcasted_iota(jnp.int32, (1, C), 1).astype(jnp.float32)

    out = jnp.full((R, 128), -jnp.inf, jnp.float32)
    for k in range(20):
        idxk = topi_ref[:, k:k + 1]

        def gath(c, g):
            off = start + c * C
            gi = off.astype(jnp.float32) + lane
            oh = (idxk == gi).astype(jnp.bfloat16)
            bh = bhi_ref[pl.ds(pl.multiple_of(off, C), C), :]
            bl = blo_ref[pl.ds(pl.multiple_of(off, C), C), :]
            g = g + _dot(oh, bh, ((1,), (0,)), precision=None)
            g = g + _dot(oh, bl, ((1,), (0,)), precision=None)
            return g

        gk = lax.fori_loop(0, ncr, gath, jnp.zeros((R, 64), jnp.float32),
                           unroll=False)
        h = jnp.maximum(a + gk, 0.0)
        out = jnp.maximum(out, _dot(h, w2_ref[:], ((1,), (0,))))

    out_ref[:] = out + b2_ref[:]


def kernel(x, batch, W1, b1, W2, b2, _debug_parts=False):
    n, d = x.shape
    n_pad = ((n + C - 1) // C) * C
    nb = n_pad // R

    pad_id = batch[-1] + 1
    x_pad = jnp.pad(x, ((0, n_pad - n), (0, 0)))
    batch_pad = jnp.concatenate(
        [batch, jnp.full((n_pad - n,), pad_id, batch.dtype)])

    x_bf = x_pad.astype(jnp.bfloat16)
    sq = jnp.sum(x_pad * x_pad, axis=1)
    sqr_f = sq[:, None]
    nch_total = n_pad // C
    sqc3 = jnp.broadcast_to(sq.reshape(nch_total, 1, C),
                            (nch_total, 8, C))

    iota = jnp.arange(n_pad, dtype=jnp.int32)
    is_start = jnp.concatenate(
        [jnp.ones((1,), bool), batch_pad[1:] != batch_pad[:-1]])
    is_end = jnp.concatenate(
        [batch_pad[1:] != batch_pad[:-1], jnp.ones((1,), bool)])
    rs_all = lax.cummax(jnp.where(is_start, iota, 0))
    re_all = lax.cummin(jnp.where(is_end, iota + 1, n_pad)[::-1])[::-1]
    start_blk = rs_all.reshape(nb, R)[:, 0].astype(jnp.int32)
    end_blk = re_all.reshape(nb, R)[:, -1].astype(jnp.int32)
    start_al = (start_blk // C) * C
    ncr = (end_blk - start_al + C - 1) // C

    rs_f = rs_all.astype(jnp.float32)[:, None]
    re_f = re_all.astype(jnp.float32)[:, None]

    W1m = W1[:d] - W1[d:]
    W1b = W1[d:]
    b1r = b1[None, :]
    b2r = b2[None, :]

    smem = pl.BlockSpec(memory_space=pltpu.SMEM)
    full = pl.BlockSpec(memory_space=pltpu.VMEM)

    grid = (nb,)
    topi, A, B = pl.pallas_call(
        _knn_proj_kernel,
        grid=grid,
        in_specs=[
            smem, smem,
            full, full,
            pl.BlockSpec((R, 1), lambda b: (b, 0)),
            full,
            pl.BlockSpec((R, 1), lambda b: (b, 0)),
            pl.BlockSpec((R, 1), lambda b: (b, 0)),
            full, full, full,
        ],
        out_specs=[
            pl.BlockSpec((R, 20), lambda b: (b, 0)),
            pl.BlockSpec((R, 64), lambda b: (b, 0)),
            pl.BlockSpec((R, 64), lambda b: (b, 0)),
        ],
        out_shape=[
            jax.ShapeDtypeStruct((n_pad, 20), jnp.float32),
            jax.ShapeDtypeStruct((n_pad, 64), jnp.float32),
            jax.ShapeDtypeStruct((n_pad, 64), jnp.float32),
        ],
        scratch_shapes=[pltpu.VMEM((R, n_pad), jnp.float32)],
    )(start_al, ncr, x_bf, x_pad, sqr_f, sqc3, rs_f, re_f, W1m, W1b, b1r)

    Bhi = B.astype(jnp.bfloat16)
    Blo = (B - Bhi.astype(jnp.float32)).astype(jnp.bfloat16)

    out = pl.pallas_call(
        _edge_mlp_kernel,
        grid=grid,
        in_specs=[
            smem, smem,
            pl.BlockSpec((R, 20), lambda b: (b, 0)),
            pl.BlockSpec((R, 64), lambda b: (b, 0)),
            full, full, full, full,
        ],
        out_specs=pl.BlockSpec((R, 128), lambda b: (b, 0)),
        out_shape=jax.ShapeDtypeStruct((n_pad, 128), jnp.float32),
    )(start_al, ncr, topi, A, Bhi, Blo, W2, b2r)

    if _debug_parts:
        return out[:n], topi, A, B
    return out[:n]


# speedup vs baseline: 12.0478x; 2.2000x over previous
import jax
import jax.numpy as jnp
from jax import lax
from jax.experimental import pallas as pl
from jax.experimental.pallas import tpu as pltpu

R = 128
C = 512
K = 20
BIG = 1e30
IDX_BIG = 1e9

HIGH = lax.Precision.HIGHEST


def _dot(a, b, dims, precision=HIGH):
    return lax.dot_general(a, b, (dims, ((), ())),
                           precision=precision,
                           preferred_element_type=jnp.float32)


def _knn_proj_kernel(starts_ref, ncr_ref, xbf_ref, xf_ref, sqc_ref, sqr_ref,
                     rs_ref, re_ref, w1m_ref, w1b_ref, b1_ref,
                     topi_ref, a_ref, b_ref, dist_scr):
    blk = pl.program_id(0)
    start = starts_ref[blk]
    ncr = ncr_ref[blk]

    xr_b = xbf_ref[pl.ds(pl.multiple_of(blk * R, R), R), :]
    rs = rs_ref[0, 0:1, :]
    re = re_ref[0, 0:1, :]
    sqr = sqr_ref[0, 0:1, :]

    xr = xf_ref[pl.ds(pl.multiple_of(blk * R, R), R), :]
    a_ref[:] = _dot(xr, w1m_ref[:], ((1,), (0,))) + b1_ref[:]
    b_ref[:] = _dot(xr, w1b_ref[:], ((1,), (0,)))

    sub = lax.broadcasted_iota(jnp.int32, (C, 1), 0).astype(jnp.float32)

    def fill(c, _):
        off = start + c * C
        xc_c = xbf_ref[pl.ds(pl.multiple_of(off, C), C), :]
        d0 = _dot(xc_c, xr_b, ((1,), (1,)), precision=None)
        sqc = sqc_ref[pl.ds(pl.multiple_of(off, C), C), :]
        d = (sqr + sqc) - 2.0 * d0
        gi = off.astype(jnp.float32) + sub
        valid = (gi >= rs) & (gi < re)
        dist_scr[pl.ds(pl.multiple_of(c * C, C), C), :] = jnp.where(valid, d, BIG)
        return 0

    lax.fori_loop(0, ncr, fill, 0, unroll=False)

    m_prev = jnp.full((1, R), -jnp.inf, jnp.float32)
    i_prev = jnp.full((1, R), -1.0, jnp.float32)
    rows = []
    for _ in range(K):
        def scan(c, carry):
            bv, bi = carry
            v = dist_scr[pl.ds(pl.multiple_of(c * C, C), C), :]
            gi = (start + c * C).astype(jnp.float32) + sub
            ok = (v > m_prev) | ((v == m_prev) & (gi > i_prev))
            vv = jnp.where(ok, v, jnp.inf)
            cm = jnp.min(vv, axis=0, keepdims=True)
            ci = jnp.min(jnp.where(vv == cm, gi, IDX_BIG), axis=0,
                         keepdims=True)
            take = (cm < bv) | ((cm == bv) & (ci < bi))
            return jnp.where(take, cm, bv), jnp.where(take, ci, bi)

        m_prev, i_prev = lax.fori_loop(
            0, ncr, scan,
            (jnp.full((1, R), jnp.inf, jnp.float32),
             jnp.full((1, R), IDX_BIG, jnp.float32)),
            unroll=False)
        rows.append(i_prev)

    topi_ref[:] = jnp.concatenate(rows, axis=0)


def _edge_mlp_kernel(starts_ref, ncr_ref, topi_ref, a_ref, bcat_ref,
                     w2_ref, b2_ref, out_ref, g_scr):
    blk = pl.program_id(0)
    start = starts_ref[blk]
    ncr = ncr_ref[blk]

    a = a_ref[:]
    sub = lax.broadcasted_iota(jnp.int32, (C, 1), 0).astype(jnp.float32)
    topi = topi_ref[:]

    g_scr[:] = jnp.zeros((R, K * 64), jnp.float32)

    def gath_chunk(c, _):
        off = start + c * C
        gi = off.astype(jnp.float32) + sub
        bc = bcat_ref[pl.ds(pl.multiple_of(off, C), C), :]
        for k in range(K):
            idxk = topi[k:k + 1, :]
            oh = (gi == idxk).astype(jnp.bfloat16)
            g2 = _dot(oh, bc, ((0,), (0,)), precision=None)
            g = g2[:, :64] + g2[:, 64:]
            g_scr[:, k * 64:(k + 1) * 64] += g
        return 0

    lax.fori_loop(0, ncr, gath_chunk, 0, unroll=False)

    out = jnp.full((R, 128), -jnp.inf, jnp.float32)
    for k in range(K):
        h = jnp.maximum(a + g_scr[:, k * 64:(k + 1) * 64], 0.0)
        out = jnp.maximum(out, _dot(h, w2_ref[:], ((1,), (0,))))

    out_ref[:] = out + b2_ref[:]


def kernel(x, batch, W1, b1, W2, b2, _debug_parts=False):
    n, d = x.shape
    n_pad = ((n + C - 1) // C) * C
    nb = n_pad // R

    pad_id = batch[-1] + 1
    x_pad = jnp.pad(x, ((0, n_pad - n), (0, 0)))
    batch_pad = jnp.concatenate(
        [batch, jnp.full((n_pad - n,), pad_id, batch.dtype)])

    x_bf = x_pad.astype(jnp.bfloat16)
    sq = jnp.sum(x_pad * x_pad, axis=1)
    sq_col = sq[:, None]

    iota = jnp.arange(n_pad, dtype=jnp.int32)
    is_start = jnp.concatenate(
        [jnp.ones((1,), bool), batch_pad[1:] != batch_pad[:-1]])
    is_end = jnp.concatenate(
        [batch_pad[1:] != batch_pad[:-1], jnp.ones((1,), bool)])
    rs_all = lax.cummax(jnp.where(is_start, iota, 0))
    re_all = lax.cummin(jnp.where(is_end, iota + 1, n_pad)[::-1])[::-1]
    start_blk = rs_all.reshape(nb, R)[:, 0].astype(jnp.int32)
    end_blk = re_all.reshape(nb, R)[:, -1].astype(jnp.int32)
    start_al = (start_blk // C) * C
    ncr = (end_blk - start_al + C - 1) // C

    def row_tiles(v):
        return jnp.broadcast_to(
            v.astype(jnp.float32).reshape(nb, 1, R), (nb, 8, R))

    rs_t = row_tiles(rs_all)
    re_t = row_tiles(re_all)
    sqr_t = row_tiles(sq)

    W1m = W1[:d] - W1[d:]
    W1b = W1[d:]
    b1r = b1[None, :]
    b2r = b2[None, :]

    smem = pl.BlockSpec(memory_space=pltpu.SMEM)
    full = pl.BlockSpec(memory_space=pltpu.VMEM)

    grid = (nb,)
    topi, A, B = pl.pallas_call(
        _knn_proj_kernel,
        grid=grid,
        in_specs=[
            smem, smem,
            full, full, full,
            pl.BlockSpec((1, 8, R), lambda b: (b, 0, 0)),
            pl.BlockSpec((1, 8, R), lambda b: (b, 0, 0)),
            pl.BlockSpec((1, 8, R), lambda b: (b, 0, 0)),
            full, full, full,
        ],
        out_specs=[
            pl.BlockSpec((K, R), lambda b: (0, b)),
            pl.BlockSpec((R, 64), lambda b: (b, 0)),
            pl.BlockSpec((R, 64), lambda b: (b, 0)),
        ],
        out_shape=[
            jax.ShapeDtypeStruct((K, n_pad), jnp.float32),
            jax.ShapeDtypeStruct((n_pad, 64), jnp.float32),
            jax.ShapeDtypeStruct((n_pad, 64), jnp.float32),
        ],
        scratch_shapes=[pltpu.VMEM((n_pad, R), jnp.float32)],
    )(start_al, ncr, x_bf, x_pad, sq_col, sqr_t, rs_t, re_t, W1m, W1b, b1r)

    Bhi = B.astype(jnp.bfloat16)
    Blo = (B - Bhi.astype(jnp.float32)).astype(jnp.bfloat16)
    Bcat = jnp.concatenate([Bhi, Blo], axis=1)

    out = pl.pallas_call(
        _edge_mlp_kernel,
        grid=grid,
        in_specs=[
            smem, smem,
            pl.BlockSpec((K, R), lambda b: (0, b)),
            pl.BlockSpec((R, 64), lambda b: (b, 0)),
            full, full, full,
        ],
        out_specs=pl.BlockSpec((R, 128), lambda b: (b, 0)),
        out_shape=jax.ShapeDtypeStruct((n_pad, 128), jnp.float32),
        scratch_shapes=[pltpu.VMEM((R, K * 64), jnp.float32)],
    )(start_al, ncr, topi, A, Bcat, W2, b2r)

    if _debug_parts:
        return out[:n], topi.T, A, B
    return out[:n]


# speedup vs baseline: 14.3541x; 1.1914x over previous
import jax
import jax.numpy as jnp
from jax import lax
from jax.experimental import pallas as pl
from jax.experimental.pallas import tpu as pltpu

R = 128
C = 512
K = 20
BIG = 1e30
IDX_BIG = 1e9

HIGH = lax.Precision.HIGHEST


def _dot(a, b, dims, precision=HIGH):
    return lax.dot_general(a, b, (dims, ((), ())),
                           precision=precision,
                           preferred_element_type=jnp.float32)


def _knn_proj_kernel(starts_ref, ncr_ref, xbf_ref, xf_ref, sqc_ref, sqr_ref,
                     rs_ref, re_ref, w1m_ref, w1b_ref, b1_ref,
                     topi_ref, a_ref, b_ref, dist_scr):
    blk = pl.program_id(0)
    start = starts_ref[blk]
    ncr = ncr_ref[blk]

    xr_b = xbf_ref[pl.ds(pl.multiple_of(blk * R, R), R), :]
    rs = rs_ref[0, 0:1, :]
    re = re_ref[0, 0:1, :]
    sqr = sqr_ref[0, 0:1, :]

    xr = xf_ref[pl.ds(pl.multiple_of(blk * R, R), R), :]
    a_ref[:] = _dot(xr, w1m_ref[:], ((1,), (0,))) + b1_ref[:]
    b_ref[:] = _dot(xr, w1b_ref[:], ((1,), (0,)))

    sub = lax.broadcasted_iota(jnp.int32, (C, 1), 0).astype(jnp.float32)

    def fill(c, _):
        off = start + c * C
        xc_c = xbf_ref[pl.ds(pl.multiple_of(off, C), C), :]
        d0 = _dot(xc_c, xr_b, ((1,), (1,)), precision=None)
        sqc = sqc_ref[pl.ds(pl.multiple_of(off, C), C), :]
        d = (sqr + sqc) - 2.0 * d0
        gi = off.astype(jnp.float32) + sub
        valid = (gi >= rs) & (gi < re)
        dist_scr[pl.ds(pl.multiple_of(c * C, C), C), :] = jnp.where(valid, d, BIG)
        return 0

    lax.fori_loop(0, ncr, fill, 0, unroll=False)

    m_prev = jnp.full((1, R), -jnp.inf, jnp.float32)
    i_prev = jnp.full((1, R), -1.0, jnp.float32)
    rows = []
    for _ in range(K):
        def scan(c, carry):
            bv, bi = carry
            v = dist_scr[pl.ds(pl.multiple_of(c * C, C), C), :]
            gi = (start + c * C).astype(jnp.float32) + sub
            ok = (v > m_prev) | ((v == m_prev) & (gi > i_prev))
            vv = jnp.where(ok, v, jnp.inf)
            cm = jnp.min(vv, axis=0, keepdims=True)
            ci = jnp.min(jnp.where(vv == cm, gi, IDX_BIG), axis=0,
                         keepdims=True)
            take = (cm < bv) | ((cm == bv) & (ci < bi))
            return jnp.where(take, cm, bv), jnp.where(take, ci, bi)

        m_prev, i_prev = lax.fori_loop(
            0, ncr, scan,
            (jnp.full((1, R), jnp.inf, jnp.float32),
             jnp.full((1, R), IDX_BIG, jnp.float32)),
            unroll=False)
        rows.append(i_prev)

    topi_ref[0, 0:1, :] = jnp.concatenate(rows, axis=1)


def _edge_mlp_kernel(starts_ref, ncr_ref, topi_ref, a_ref, bhi_ref,
                     w2_ref, b2_ref, out_ref, g_scr):
    blk = pl.program_id(0)
    start = starts_ref[blk]
    ncr = ncr_ref[blk]

    a = a_ref[:]
    sub = lax.broadcasted_iota(jnp.int32, (C, 1), 0).astype(jnp.float32)
    tr = topi_ref[0, 0:1, :]

    for k in range(K):
        g_scr[pl.ds(k * R, R), :] = a

    def gath_chunk(c, _):
        off = start + c * C
        gi = off.astype(jnp.float32) + sub
        oh = (gi == tr).astype(jnp.bfloat16)
        bh = bhi_ref[pl.ds(pl.multiple_of(off, C), C), :]
        g_scr[:] += _dot(oh, bh, ((0,), (0,)), precision=None)
        return 0

    lax.fori_loop(0, ncr, gath_chunk, 0, unroll=False)

    h = jnp.maximum(g_scr[:], 0.0)
    o2 = _dot(h, w2_ref[:], ((1,), (0,)))
    out = o2[0:R, :]
    for k in range(1, K):
        out = jnp.maximum(out, o2[k * R:(k + 1) * R, :])

    out_ref[:] = out + b2_ref[:]


def kernel(x, batch, W1, b1, W2, b2, _debug_parts=False):
    n, d = x.shape
    n_pad = ((n + C - 1) // C) * C
    nb = n_pad // R

    pad_id = batch[-1] + 1
    x_pad = jnp.pad(x, ((0, n_pad - n), (0, 0)))
    batch_pad = jnp.concatenate(
        [batch, jnp.full((n_pad - n,), pad_id, batch.dtype)])

    x_bf = x_pad.astype(jnp.bfloat16)
    sq = jnp.sum(x_pad * x_pad, axis=1)
    sq_col = sq[:, None]

    iota = jnp.arange(n_pad, dtype=jnp.int32)
    is_start = jnp.concatenate(
        [jnp.ones((1,), bool), batch_pad[1:] != batch_pad[:-1]])
    is_end = jnp.concatenate(
        [batch_pad[1:] != batch_pad[:-1], jnp.ones((1,), bool)])
    rs_all = lax.cummax(jnp.where(is_start, iota, 0))
    re_all = lax.cummin(jnp.where(is_end, iota + 1, n_pad)[::-1])[::-1]
    start_blk = rs_all.reshape(nb, R)[:, 0].astype(jnp.int32)
    end_blk = re_all.reshape(nb, R)[:, -1].astype(jnp.int32)
    start_al = (start_blk // C) * C
    ncr = (end_blk - start_al + C - 1) // C

    def row_tiles(v):
        return jnp.broadcast_to(
            v.astype(jnp.float32).reshape(nb, 1, R), (nb, 8, R))

    rs_t = row_tiles(rs_all)
    re_t = row_tiles(re_all)
    sqr_t = row_tiles(sq)

    W1m = W1[:d] - W1[d:]
    W1b = W1[d:]
    b1r = b1[None, :]
    b2r = b2[None, :]

    smem = pl.BlockSpec(memory_space=pltpu.SMEM)
    full = pl.BlockSpec(memory_space=pltpu.VMEM)

    grid = (nb,)
    topi, A, B = pl.pallas_call(
        _knn_proj_kernel,
        grid=grid,
        in_specs=[
            smem, smem,
            full, full, full,
            pl.BlockSpec((1, 8, R), lambda b: (b, 0, 0)),
            pl.BlockSpec((1, 8, R), lambda b: (b, 0, 0)),
            pl.BlockSpec((1, 8, R), lambda b: (b, 0, 0)),
            full, full, full,
        ],
        out_specs=[
            pl.BlockSpec((1, 8, K * R), lambda b: (b, 0, 0)),
            pl.BlockSpec((R, 64), lambda b: (b, 0)),
            pl.BlockSpec((R, 64), lambda b: (b, 0)),
        ],
        out_shape=[
            jax.ShapeDtypeStruct((nb, 8, K * R), jnp.float32),
            jax.ShapeDtypeStruct((n_pad, 64), jnp.float32),
            jax.ShapeDtypeStruct((n_pad, 64), jnp.float32),
        ],
        scratch_shapes=[pltpu.VMEM((n_pad, R), jnp.float32)],
    )(start_al, ncr, x_bf, x_pad, sq_col, sqr_t, rs_t, re_t, W1m, W1b, b1r)

    Bhi = B.astype(jnp.bfloat16)

    out = pl.pallas_call(
        _edge_mlp_kernel,
        grid=grid,
        in_specs=[
            smem, smem,
            pl.BlockSpec((1, 8, K * R), lambda b: (b, 0, 0)),
            pl.BlockSpec((R, 64), lambda b: (b, 0)),
            full, full, full,
        ],
        out_specs=pl.BlockSpec((R, 128), lambda b: (b, 0)),
        out_shape=jax.ShapeDtypeStruct((n_pad, 128), jnp.float32),
        scratch_shapes=[pltpu.VMEM((K * R, 64), jnp.float32)],
    )(start_al, ncr, topi, A, Bhi, W2, b2r)

    if _debug_parts:
        topi_nk = topi[:, 0, :].reshape(nb, K, R).transpose(0, 2, 1)
        return out[:n], topi_nk.reshape(n_pad, K), A, B
    return out[:n]


# speedup vs baseline: 15.1079x; 1.0525x over previous
import jax
import jax.numpy as jnp
from jax import lax
from jax.experimental import pallas as pl
from jax.experimental.pallas import tpu as pltpu

R = 256
C = 512
K = 20
BIG = 1e30
IDX_BIG = 1e9

HIGH = lax.Precision.HIGHEST


def _dot(a, b, dims, precision=HIGH):
    return lax.dot_general(a, b, (dims, ((), ())),
                           precision=precision,
                           preferred_element_type=jnp.float32)


def _knn_proj_kernel(starts_ref, ncr_ref, xbf_ref, xf_ref, sqc_ref, sqr_ref,
                     rs_ref, re_ref, w1m_ref, w1b_ref, b1_ref,
                     topi_ref, a_ref, b_ref, dist_scr):
    blk = pl.program_id(0)
    start = starts_ref[blk]
    ncr = ncr_ref[blk]

    xr_b = xbf_ref[pl.ds(pl.multiple_of(blk * R, R), R), :]
    rs = rs_ref[0, 0:1, :]
    re = re_ref[0, 0:1, :]
    sqr = sqr_ref[0, 0:1, :]

    xr = xf_ref[pl.ds(pl.multiple_of(blk * R, R), R), :]
    a_ref[:] = _dot(xr, w1m_ref[:], ((1,), (0,))) + b1_ref[:]
    b_ref[:] = _dot(xr, w1b_ref[:], ((1,), (0,)))

    sub = lax.broadcasted_iota(jnp.int32, (C, 1), 0).astype(jnp.float32)

    def fill(c, _):
        off = start + c * C
        xc_c = xbf_ref[pl.ds(pl.multiple_of(off, C), C), :]
        d0 = _dot(xc_c, xr_b, ((1,), (1,)), precision=None)
        sqc = sqc_ref[pl.ds(pl.multiple_of(off, C), C), :]
        d = (sqr + sqc) - 2.0 * d0
        gi = off.astype(jnp.float32) + sub
        valid = (gi >= rs) & (gi < re)
        dist_scr[pl.ds(pl.multiple_of(c * C, C), C), :] = jnp.where(valid, d, BIG)
        return 0

    lax.fori_loop(0, ncr, fill, 0, unroll=False)

    m_prev = jnp.full((1, R), -jnp.inf, jnp.float32)
    i_prev = jnp.full((1, R), -1.0, jnp.float32)
    rows = []
    for _ in range(K):
        def scan(c, carry):
            bv, bi = carry
            v = dist_scr[pl.ds(pl.multiple_of(c * C, C), C), :]
            gi = (start + c * C).astype(jnp.float32) + sub
            ok = (v > m_prev) | ((v == m_prev) & (gi > i_prev))
            vv = jnp.where(ok, v, jnp.inf)
            cm = jnp.min(vv, axis=0, keepdims=True)
            ci = jnp.min(jnp.where(vv == cm, gi, IDX_BIG), axis=0,
                         keepdims=True)
            take = (cm < bv) | ((cm == bv) & (ci < bi))
            return jnp.where(take, cm, bv), jnp.where(take, ci, bi)

        m_prev, i_prev = lax.fori_loop(
            0, ncr, scan,
            (jnp.full((1, R), jnp.inf, jnp.float32),
             jnp.full((1, R), IDX_BIG, jnp.float32)),
            unroll=False)
        rows.append(i_prev)

    topi_ref[0, 0:1, :] = jnp.concatenate(rows, axis=1)


def _edge_mlp_kernel(starts_ref, ncr_ref, topi_ref, a_ref, bhi_ref,
                     w2_ref, b2_ref, out_ref, g_scr):
    blk = pl.program_id(0)
    start = starts_ref[blk]
    ncr = ncr_ref[blk]

    a = a_ref[:]
    sub = lax.broadcasted_iota(jnp.int32, (C, 1), 0).astype(jnp.float32)
    tr = topi_ref[0, 0:1, :]

    for k in range(K):
        g_scr[pl.ds(k * R, R), :] = a

    def gath_chunk(c, _):
        off = start + c * C
        gi = off.astype(jnp.float32) + sub
        oh = (gi == tr).astype(jnp.bfloat16)
        bh = bhi_ref[pl.ds(pl.multiple_of(off, C), C), :]
        g_scr[:] += _dot(oh, bh, ((0,), (0,)), precision=None)
        return 0

    lax.fori_loop(0, ncr, gath_chunk, 0, unroll=False)

    h = jnp.maximum(g_scr[:], 0.0)
    o2 = _dot(h, w2_ref[:], ((1,), (0,)))
    out = o2[0:R, :]
    for k in range(1, K):
        out = jnp.maximum(out, o2[k * R:(k + 1) * R, :])

    out_ref[:] = out + b2_ref[:]


def kernel(x, batch, W1, b1, W2, b2, _debug_parts=False):
    n, d = x.shape
    n_pad = ((n + C - 1) // C) * C
    nb = n_pad // R

    pad_id = batch[-1] + 1
    x_pad = jnp.pad(x, ((0, n_pad - n), (0, 0)))
    batch_pad = jnp.concatenate(
        [batch, jnp.full((n_pad - n,), pad_id, batch.dtype)])

    x_bf = x_pad.astype(jnp.bfloat16)
    sq = jnp.sum(x_pad * x_pad, axis=1)
    sq_col = sq[:, None]

    iota = jnp.arange(n_pad, dtype=jnp.int32)
    is_start = jnp.concatenate(
        [jnp.ones((1,), bool), batch_pad[1:] != batch_pad[:-1]])
    is_end = jnp.concatenate(
        [batch_pad[1:] != batch_pad[:-1], jnp.ones((1,), bool)])
    rs_all = lax.cummax(jnp.where(is_start, iota, 0))
    re_all = lax.cummin(jnp.where(is_end, iota + 1, n_pad)[::-1])[::-1]
    start_blk = rs_all.reshape(nb, R)[:, 0].astype(jnp.int32)
    end_blk = re_all.reshape(nb, R)[:, -1].astype(jnp.int32)
    start_al = (start_blk // C) * C
    ncr = (end_blk - start_al + C - 1) // C

    def row_tiles(v):
        return jnp.broadcast_to(
            v.astype(jnp.float32).reshape(nb, 1, R), (nb, 8, R))

    rs_t = row_tiles(rs_all)
    re_t = row_tiles(re_all)
    sqr_t = row_tiles(sq)

    W1m = W1[:d] - W1[d:]
    W1b = W1[d:]
    b1r = b1[None, :]
    b2r = b2[None, :]

    smem = pl.BlockSpec(memory_space=pltpu.SMEM)
    full = pl.BlockSpec(memory_space=pltpu.VMEM)

    grid = (nb,)
    topi, A, B = pl.pallas_call(
        _knn_proj_kernel,
        grid=grid,
        in_specs=[
            smem, smem,
            full, full, full,
            pl.BlockSpec((1, 8, R), lambda b: (b, 0, 0)),
            pl.BlockSpec((1, 8, R), lambda b: (b, 0, 0)),
            pl.BlockSpec((1, 8, R), lambda b: (b, 0, 0)),
            full, full, full,
        ],
        out_specs=[
            pl.BlockSpec((1, 8, K * R), lambda b: (b, 0, 0)),
            pl.BlockSpec((R, 64), lambda b: (b, 0)),
            pl.BlockSpec((R, 64), lambda b: (b, 0)),
        ],
        out_shape=[
            jax.ShapeDtypeStruct((nb, 8, K * R), jnp.float32),
            jax.ShapeDtypeStruct((n_pad, 64), jnp.float32),
            jax.ShapeDtypeStruct((n_pad, 64), jnp.float32),
        ],
        scratch_shapes=[pltpu.VMEM((n_pad, R), jnp.float32)],
    )(start_al, ncr, x_bf, x_pad, sq_col, sqr_t, rs_t, re_t, W1m, W1b, b1r)

    Bhi = B.astype(jnp.bfloat16)

    out = pl.pallas_call(
        _edge_mlp_kernel,
        grid=grid,
        in_specs=[
            smem, smem,
            pl.BlockSpec((1, 8, K * R), lambda b: (b, 0, 0)),
            pl.BlockSpec((R, 64), lambda b: (b, 0)),
            full, full, full,
        ],
        out_specs=pl.BlockSpec((R, 128), lambda b: (b, 0)),
        out_shape=jax.ShapeDtypeStruct((n_pad, 128), jnp.float32),
        scratch_shapes=[pltpu.VMEM((K * R, 64), jnp.float32)],
    )(start_al, ncr, topi, A, Bhi, W2, b2r)

    if _debug_parts:
        topi_nk = topi[:, 0, :].reshape(nb, K, R).transpose(0, 2, 1)
        return out[:n], topi_nk.reshape(n_pad, K), A, B
    return out[:n]
